# Initial kernel scaffold; baseline (speedup 1.0000x reference)
import jax
import jax.numpy as jnp
from jax.experimental import pallas as pl


def kernel(pos_s, pos_e, pe_ss, pe_ee, W, b):
    raise NotImplementedError("write your pallas kernel here")


# speedup vs baseline: 3.7620x; 3.7620x over previous
import functools

import jax
import jax.numpy as jnp
from jax import lax
from jax.experimental import pallas as pl
from jax.experimental.pallas import tpu as pltpu
from jax.experimental.pallas import tpu_sc as plsc

MAX_SEQ_LEN = 512
H = 128
B, S = 4, 256
TABLE_ROWS = 2 * MAX_SEQ_LEN + 1

NC, NS, L = 2, 16, 16
NW = NC * NS

UNITS = B * S
UNITS_PER_W = UNITS // NW
CHUNK = 128


def _proj_body(pe_ss_ref, pe_ee_ref, w_ref, b_ref, pss_ref, pee_ref):
    dn = (((1,), (1,)), ((), ()))
    w1 = w_ref[:, :H]
    w2 = w_ref[:, H:]
    pss_ref[...] = lax.dot_general(
        pe_ss_ref[...], w1, dn, preferred_element_type=jnp.float32)
    pee_ref[...] = lax.dot_general(
        pe_ee_ref[...], w2, dn, preferred_element_type=jnp.float32) + b_ref[...]


def _project(pe_ss, pe_ee, W, b2d):
    return pl.pallas_call(
        _proj_body,
        out_shape=(
            jax.ShapeDtypeStruct((TABLE_ROWS, H), jnp.float32),
            jax.ShapeDtypeStruct((TABLE_ROWS, H), jnp.float32),
        ),
    )(pe_ss, pe_ee, W, b2d)


def _sc_body(pos_s_hbm, pos_e_hbm, pss_hbm, pee_hbm, out_hbm,
             pos_s_v, pos_e_v, idx_ss_v, idx_ee_v, rows_ss, rows_ee, sem):
    wid = lax.axis_index("s") * NC + lax.axis_index("c")
    b = wid // (NW // B)
    i_base = (wid % (NW // B)) * UNITS_PER_W

    pltpu.sync_copy(pos_s_hbm.at[b], pos_s_v.at[pl.ds(0, S)])
    pltpu.sync_copy(pos_e_hbm.at[b], pos_e_v.at[pl.ds(0, S)])

    def unit_body(u, carry):
        i = i_base + u
        u_global = wid * UNITS_PER_W + u
        bs = jnp.full((L,), pos_s_v[pl.ds(i, L)][0] + MAX_SEQ_LEN, jnp.int32)
        be = jnp.full((L,), pos_e_v[pl.ds(i, L)][0] + MAX_SEQ_LEN, jnp.int32)
        for jc in range(S // CHUNK):
            for v in range(CHUNK // L):
                off = jc * CHUNK + v * L
                idx_ss_v[pl.ds(v * L, L)] = bs - pos_s_v[pl.ds(off, L)]
                idx_ee_v[pl.ds(v * L, L)] = be - pos_e_v[pl.ds(off, L)]
            cs = pltpu.async_copy(pss_hbm.at[idx_ss_v], rows_ss, sem)
            ce = pltpu.async_copy(pee_hbm.at[idx_ee_v], rows_ee, sem)
            cs.wait()
            ce.wait()

            def row_body(r, c2):
                for v in range(H // L):
                    sl = pl.ds(v * L, L)
                    rows_ss[r, sl] = jnp.maximum(
                        rows_ss[r, sl] + rows_ee[r, sl], 0.0)
                return c2

            lax.fori_loop(0, CHUNK, row_body, 0, unroll=2)
            row_base = u_global * S + jc * CHUNK
            pltpu.sync_copy(rows_ss, out_hbm.at[pl.ds(row_base, CHUNK)])
        return carry

    lax.fori_loop(0, UNITS_PER_W, unit_body, 0)


def _sc_gather(pos_s, pos_e, proj_ss, proj_ee):
    mesh = plsc.VectorSubcoreMesh(
        core_axis_name="c", subcore_axis_name="s",
        num_cores=NC, num_subcores=NS)
    fn = pl.kernel(
        _sc_body,
        out_type=jax.ShapeDtypeStruct((B * S * S, H), jnp.float32),
        mesh=mesh,
        scratch_types=[
            pltpu.VMEM((S + L,), jnp.int32),
            pltpu.VMEM((S + L,), jnp.int32),
            pltpu.VMEM((CHUNK,), jnp.int32),
            pltpu.VMEM((CHUNK,), jnp.int32),
            pltpu.VMEM((CHUNK, H), jnp.float32),
            pltpu.VMEM((CHUNK, H), jnp.float32),
            pltpu.SemaphoreType.DMA,
        ],
    )
    return fn(pos_s, pos_e, proj_ss, proj_ee)


def kernel(pos_s, pos_e, pe_ss, pe_ee, W, b):
    pos_s = pos_s.astype(jnp.int32)
    pos_e = pos_e.astype(jnp.int32)
    proj_ss, proj_ee = _project(pe_ss, pe_ee, W, b.reshape(1, H))
    out = _sc_gather(pos_s, pos_e, proj_ss, proj_ee)
    return out.reshape(B, S, S, H)


# speedup vs baseline: 5.7224x; 1.5211x over previous
import functools

import jax
import jax.numpy as jnp
from jax import lax
from jax.experimental import pallas as pl
from jax.experimental.pallas import tpu as pltpu
from jax.experimental.pallas import tpu_sc as plsc

MAX_SEQ_LEN = 512
H = 128
B, S = 4, 256
TABLE_ROWS = 2 * MAX_SEQ_LEN + 1

NC, NS, L = 2, 16, 16
NW = NC * NS

UNITS = B * S
UNITS_PER_W = UNITS // NW
CHUNK = 64
NBUF = S // CHUNK


def _proj_body(pe_ss_ref, pe_ee_ref, w_ref, b_ref, pss_ref, pee_ref):
    dn = (((1,), (1,)), ((), ()))
    w1 = w_ref[:, :H]
    w2 = w_ref[:, H:]
    pss_ref[...] = lax.dot_general(
        pe_ss_ref[...], w1, dn, preferred_element_type=jnp.float32)
    pee_ref[...] = lax.dot_general(
        pe_ee_ref[...], w2, dn, preferred_element_type=jnp.float32) + b_ref[...]


def _project(pe_ss, pe_ee, W, b2d):
    return pl.pallas_call(
        _proj_body,
        out_shape=(
            jax.ShapeDtypeStruct((TABLE_ROWS, H), jnp.float32),
            jax.ShapeDtypeStruct((TABLE_ROWS, H), jnp.float32),
        ),
    )(pe_ss, pe_ee, W, b2d)


def _sc_body(pos_s_hbm, pos_e_hbm, pss_hbm, pee_hbm, out_hbm,
             pos_s_v, pos_e_v,
             iss0, iss1, iss2, iss3, iee0, iee1, iee2, iee3,
             rss0, rss1, rss2, rss3, ree0, ree1, ree2, ree3,
             gs0, gs1, gs2, gs3, os0, os1, os2, os3):
    idx_ss = [iss0, iss1, iss2, iss3]
    idx_ee = [iee0, iee1, iee2, iee3]
    rows_ss = [rss0, rss1, rss2, rss3]
    rows_ee = [ree0, ree1, ree2, ree3]
    gsem = [gs0, gs1, gs2, gs3]
    osem = [os0, os1, os2, os3]

    wid = lax.axis_index("s") * NC + lax.axis_index("c")
    b = wid // (NW // B)
    i_base = (wid % (NW // B)) * UNITS_PER_W

    pltpu.sync_copy(pos_s_hbm.at[b], pos_s_v.at[pl.ds(0, S)])
    pltpu.sync_copy(pos_e_hbm.at[b], pos_e_v.at[pl.ds(0, S)])

    def unit_bases(u):
        i = i_base + u
        bs = jnp.full((L,), pos_s_v[pl.ds(i, L)][0] + MAX_SEQ_LEN, jnp.int32)
        be = jnp.full((L,), pos_e_v[pl.ds(i, L)][0] + MAX_SEQ_LEN, jnp.int32)
        return bs, be

    def fill_idx_and_fire(bs, be, jc, bufi):
        for v in range(CHUNK // L):
            off = jc * CHUNK + v * L
            idx_ss[bufi][pl.ds(v * L, L)] = bs - pos_s_v[pl.ds(off, L)]
            idx_ee[bufi][pl.ds(v * L, L)] = be - pos_e_v[pl.ds(off, L)]
        pltpu.async_copy(pss_hbm.at[idx_ss[bufi]], rows_ss[bufi], gsem[bufi])
        pltpu.async_copy(pee_hbm.at[idx_ee[bufi]], rows_ee[bufi], gsem[bufi])

    def wait_gather(bufi):
        pltpu.make_async_copy(
            pss_hbm.at[idx_ss[bufi]], rows_ss[bufi], gsem[bufi]).wait()
        pltpu.make_async_copy(
            pee_hbm.at[idx_ee[bufi]], rows_ee[bufi], gsem[bufi]).wait()

    def wait_out(bufi):
        pltpu.make_async_copy(
            rows_ss[bufi], out_hbm.at[pl.ds(0, CHUNK)], osem[bufi]).wait()

    bs0, be0 = unit_bases(0)
    fill_idx_and_fire(bs0, be0, 0, 0)

    def unit_body(u, carry):
        bs, be = unit_bases(u)
        for bufi in range(NBUF):
            nb = (bufi + 1) % NBUF
            if bufi == NBUF - 1:
                wait_out(nb)
            else:
                @pl.when(u > 0)
                def _():
                    wait_out(nb)
            if bufi < NBUF - 1:
                fill_idx_and_fire(bs, be, bufi + 1, nb)
            else:
                @pl.when(u + 1 < UNITS_PER_W)
                def _():
                    nbs, nbe = unit_bases(u + 1)
                    fill_idx_and_fire(nbs, nbe, 0, nb)
            wait_gather(bufi)

            def row_body(r, c2):
                for v in range(H // L):
                    sl = pl.ds(v * L, L)
                    rows_ss[bufi][r, sl] = jnp.maximum(
                        rows_ss[bufi][r, sl] + rows_ee[bufi][r, sl], 0.0)
                return c2

            lax.fori_loop(0, CHUNK, row_body, 0, unroll=2)
            row_base = (wid * UNITS_PER_W + u) * S + bufi * CHUNK
            pltpu.async_copy(
                rows_ss[bufi], out_hbm.at[pl.ds(row_base, CHUNK)], osem[bufi])
        return carry

    lax.fori_loop(0, UNITS_PER_W, unit_body, 0)

    for bufi in range(1, NBUF):
        wait_out(bufi)


def _sc_gather(pos_s, pos_e, proj_ss, proj_ee):
    mesh = plsc.VectorSubcoreMesh(
        core_axis_name="c", subcore_axis_name="s",
        num_cores=NC, num_subcores=NS)
    fn = pl.kernel(
        _sc_body,
        out_type=jax.ShapeDtypeStruct((B * S * S, H), jnp.float32),
        mesh=mesh,
        scratch_types=(
            [pltpu.VMEM((S + L,), jnp.int32)] * 2
            + [pltpu.VMEM((CHUNK,), jnp.int32)] * (2 * NBUF)
            + [pltpu.VMEM((CHUNK, H), jnp.float32)] * (2 * NBUF)
            + [pltpu.SemaphoreType.DMA] * (2 * NBUF)
        ),
    )
    return fn(pos_s, pos_e, proj_ss, proj_ee)


def kernel(pos_s, pos_e, pe_ss, pe_ee, W, b):
    pos_s = pos_s.astype(jnp.int32)
    pos_e = pos_e.astype(jnp.int32)
    proj_ss, proj_ee = _project(pe_ss, pe_ee, W, b.reshape(1, H))
    out = _sc_gather(pos_s, pos_e, proj_ss, proj_ee)
    return out.reshape(B, S, S, H)
